# baseline (speedup 1.0000x reference)
import jax
import jax.numpy as jnp
from jax import lax
from jax.experimental import pallas as pl
from jax.experimental.pallas import tpu as pltpu
from jax.experimental.pallas import tpu_sc as plsc

NW = 32
VOCAB = 1000000
D = 64
NFULL = 7812
SEQ = 50
BATCH = 16384
NBLK = (BATCH // 128) * SEQ


def _relayout_body(tt_ref, r5_ref, bin_ref, bout_ref, tin_ref, tout_ref,
                   gsem, osem):
    wid = lax.axis_index("s") * 2 + lax.axis_index("c")
    iota = lax.iota(jnp.int32, 16)
    rowvec = [(iota + 16 * j) >> 1 for j in range(8)]
    parity64 = (iota & 1) << 6

    n_k = (NFULL - wid + 31) // 32

    def body(k, carry):
        vt = wid + 32 * k
        copies = []
        for dt in range(8):
            copies.append(pltpu.async_copy(
                tt_ref.at[pl.ds(8 * dt, 8), pl.ds(vt * 128, 128)],
                bin_ref.at[dt], gsem))
        for c in copies:
            c.wait()
        for dt in range(8):
            for dr in range(8):
                colvec = parity64 + (8 * dt + dr)
                for j in range(8):
                    plsc.store_scatter(
                        bout_ref, [rowvec[j], colvec],
                        bin_ref[dt, dr, pl.ds(16 * j, 16)])
        pltpu.sync_copy(bout_ref, r5_ref.at[pl.ds(vt * 64, 64), :])
        return carry

    lax.fori_loop(0, n_k, body, 0)

    @pl.when(wid == NW - 1)
    def _tail():
        copies = []
        for d in range(D):
            copies.append(pltpu.async_copy(
                tt_ref.at[d, pl.ds(NFULL * 128, 64)], tin_ref.at[d], osem))
        for c in copies:
            c.wait()
        for d in range(D):
            colvec = parity64 + d
            for j in range(4):
                plsc.store_scatter(
                    tout_ref, [rowvec[j], colvec],
                    tin_ref[d, pl.ds(16 * j, 16)])
        pltpu.sync_copy(tout_ref, r5_ref.at[pl.ds(NFULL * 64, 32), :])


def _gather_body(r5_ref, idx_ref, out_ref, idxv_ref, qv_ref, staged_ref,
                 obuf_ref, gsem, osem):
    wid = lax.axis_index("s") * 2 + lax.axis_index("c")
    iota = lax.iota(jnp.int32, 16)
    rowvec = [iota + 16 * j for j in range(8)]
    per_w = NBLK // NW

    def body(k, carry):
        blk = wid * per_w + k
        s = blk // 128
        bt = blk - s * 128
        pltpu.sync_copy(idx_ref.at[blk], idxv_ref)
        parities = []
        for j in range(8):
            v = idxv_ref[pl.ds(16 * j, 16)]
            qv_ref[pl.ds(16 * j, 16)] = v >> 1
            parities.append((v & 1) << 6)
        pltpu.async_copy(r5_ref.at[qv_ref], staged_ref, gsem).wait()
        for dt in range(8):
            for dr in range(8):
                d = 8 * dt + dr
                for j in range(8):
                    obuf_ref[dr, pl.ds(16 * j, 16)] = plsc.load_gather(
                        staged_ref, [rowvec[j], parities[j] + d])
            pltpu.sync_copy(obuf_ref, out_ref.at[s, dt, bt])
        return carry

    lax.fori_loop(0, per_w, body, 0)


def kernel(data, table):
    mesh = plsc.VectorSubcoreMesh(core_axis_name="c", subcore_axis_name="s")
    params = pltpu.CompilerParams(use_tc_tiling_on_sc=True,
                                  needs_layout_passes=False)

    r5 = pl.kernel(
        _relayout_body,
        out_type=jax.ShapeDtypeStruct((VOCAB // 2, 128), jnp.float32),
        mesh=mesh,
        compiler_params=params,
        scratch_types=[
            pltpu.VMEM((8, 8, 128), jnp.float32),
            pltpu.VMEM((64, 128), jnp.float32),
            pltpu.VMEM((64, 64), jnp.float32),
            pltpu.VMEM((32, 128), jnp.float32),
            pltpu.SemaphoreType.DMA,
            pltpu.SemaphoreType.DMA,
        ],
    )(table.T)

    idx5 = data.T.reshape(NBLK, 128)
    out5 = pl.kernel(
        _gather_body,
        out_type=jax.ShapeDtypeStruct((SEQ, 8, 128, 8, 128), jnp.float32),
        mesh=mesh,
        compiler_params=params,
        scratch_types=[
            pltpu.VMEM((128,), jnp.int32),
            pltpu.VMEM((128,), jnp.int32),
            pltpu.VMEM((128, 128), jnp.float32),
            pltpu.VMEM((8, 128), jnp.float32),
            pltpu.SemaphoreType.DMA,
            pltpu.SemaphoreType.DMA,
        ],
    )(r5, idx5)

    return out5.transpose(2, 4, 0, 1, 3).reshape(BATCH, SEQ, D)
